# baseline (speedup 1.0000x reference)
import functools

import jax
import jax.numpy as jnp
from jax import lax
from jax.experimental import pallas as pl
from jax.experimental.pallas import tpu as pltpu
from jax.experimental.pallas import tpu_sc as plsc

D_MODEL = 768
NUM_WORKERS = 32
LANES = 16


def _sc_add_kernel(x_hbm, pe_hbm, o_hbm, xb, pb, sem_in, sem_out, *, rows,
                   pe_rows, chunk_rows):
    w = lax.axis_index("s") * 2 + lax.axis_index("c")
    rpw = rows // NUM_WORKERS
    ce = chunk_rows * D_MODEL
    n_chunks = rpw // chunk_rows
    row_base = w * rpw
    pe_row_base = lax.rem(row_base, pe_rows)

    for k in range(n_chunks):
        off = (row_base + k * chunk_rows) * D_MODEL
        peoff = (pe_row_base + k * chunk_rows) * D_MODEL
        pltpu.sync_copy(x_hbm.at[pl.ds(off, ce)], xb.at[0])
        pltpu.sync_copy(pe_hbm.at[pl.ds(peoff, ce)], pb.at[0])

        def body(i, _):
            sl = pl.ds(i * LANES, LANES)
            xb[0, sl] = xb[0, sl] + pb[0, sl]
            return 0

        lax.fori_loop(0, ce // LANES, body, 0)
        pltpu.sync_copy(xb.at[0], o_hbm.at[pl.ds(off, ce)])


def kernel(x, pe):
    B, S, D = x.shape
    rows = B * S
    pe_rows = pe.shape[0]
    chunk_rows = 32
    ce = chunk_rows * D

    mesh = plsc.VectorSubcoreMesh(core_axis_name="c", subcore_axis_name="s")
    sc = pl.kernel(
        functools.partial(_sc_add_kernel, rows=rows, pe_rows=pe_rows,
                          chunk_rows=chunk_rows),
        out_type=jax.ShapeDtypeStruct((rows * D,), jnp.float32),
        mesh=mesh,
        scratch_types=[
            pltpu.VMEM((2, ce), jnp.float32),
            pltpu.VMEM((2, ce), jnp.float32),
            pltpu.SemaphoreType.DMA,
            pltpu.SemaphoreType.DMA,
        ],
    )
    out = sc(x.reshape(rows * D), pe.reshape(pe_rows * D))
    return out.reshape(B, S, D)


# speedup vs baseline: 1.5582x; 1.5582x over previous
import functools

import jax
import jax.numpy as jnp
from jax import lax
from jax.experimental import pallas as pl
from jax.experimental.pallas import tpu as pltpu
from jax.experimental.pallas import tpu_sc as plsc

D_MODEL = 768
NUM_WORKERS = 32
LANES = 16


def _sc_add_kernel(x_hbm, pe_hbm, o_hbm, xb, pb,
                   sx0, sx1, sp0, sp1, so0, so1, *,
                   rows, pe_rows, chunk_rows):
    sx = (sx0, sx1)
    sp = (sp0, sp1)
    so = (so0, so1)
    w = lax.axis_index("s") * 2 + lax.axis_index("c")
    rpw = rows // NUM_WORKERS
    ce = chunk_rows * D_MODEL
    n_chunks = rpw // chunk_rows
    row_base = w * rpw
    pe_row_base = lax.rem(row_base, pe_rows)

    def in_copies(k, slot):
        off = (row_base + k * chunk_rows) * D_MODEL
        peoff = (pe_row_base + k * chunk_rows) * D_MODEL
        return (
            pltpu.make_async_copy(x_hbm.at[pl.ds(off, ce)], xb.at[slot], sx[slot]),
            pltpu.make_async_copy(pe_hbm.at[pl.ds(peoff, ce)], pb.at[slot], sp[slot]),
        )

    def out_copy(k, slot):
        off = (row_base + k * chunk_rows) * D_MODEL
        return pltpu.make_async_copy(xb.at[slot], o_hbm.at[pl.ds(off, ce)], so[slot])

    for c in in_copies(0, 0):
        c.start()
    for k in range(n_chunks):
        slot = k % 2
        other = 1 - slot
        if k + 1 < n_chunks:
            if k >= 1:
                out_copy(k - 1, other).wait()
            for c in in_copies(k + 1, other):
                c.start()
        for c in in_copies(k, slot):
            c.wait()

        @plsc.parallel_loop(0, ce // LANES, unroll=8)
        def _add(i):
            sl = pl.ds(i * LANES, LANES)
            xb[slot, sl] = xb[slot, sl] + pb[slot, sl]

        out_copy(k, slot).start()
    if n_chunks >= 2:
        out_copy(n_chunks - 2, (n_chunks - 2) % 2).wait()
    out_copy(n_chunks - 1, (n_chunks - 1) % 2).wait()


def kernel(x, pe):
    B, S, D = x.shape
    rows = B * S
    pe_rows = pe.shape[0]
    chunk_rows = 32
    ce = chunk_rows * D

    mesh = plsc.VectorSubcoreMesh(core_axis_name="c", subcore_axis_name="s")
    sc = pl.kernel(
        functools.partial(_sc_add_kernel, rows=rows, pe_rows=pe_rows,
                          chunk_rows=chunk_rows),
        out_type=jax.ShapeDtypeStruct((rows * D,), jnp.float32),
        mesh=mesh,
        scratch_types=[
            pltpu.VMEM((2, ce), jnp.float32),
            pltpu.VMEM((2, ce), jnp.float32),
        ] + [pltpu.SemaphoreType.DMA] * 6,
    )
    out = sc(x.reshape(rows * D), pe.reshape(pe_rows * D))
    return out.reshape(B, S, D)


# speedup vs baseline: 3.8153x; 2.4485x over previous
import functools

import jax
import jax.numpy as jnp
from jax import lax
from jax.experimental import pallas as pl
from jax.experimental.pallas import tpu as pltpu
from jax.experimental.pallas import tpu_sc as plsc

D_MODEL = 768
NUM_WORKERS = 32
LANES = 16


def _sc_add_kernel(x_hbm, pe_hbm, o_hbm, xb, pb,
                   sx0, sx1, sp0, sp1, so0, so1, *,
                   rows, pe_rows, chunk_rows):
    sx = (sx0, sx1)
    sp = (sp0, sp1)
    so = (so0, so1)
    w = lax.axis_index("s") * 2 + lax.axis_index("c")
    rpw = rows // NUM_WORKERS
    ce = chunk_rows * D_MODEL
    n_chunks = rpw // chunk_rows
    row_base = w * rpw
    pe_row_base = lax.rem(row_base, pe_rows)

    def in_copies(k, slot):
        r0 = row_base + k * chunk_rows
        pr0 = pe_row_base + k * chunk_rows
        return (
            pltpu.make_async_copy(x_hbm.at[pl.ds(r0, chunk_rows)], xb.at[slot], sx[slot]),
            pltpu.make_async_copy(pe_hbm.at[pl.ds(pr0, chunk_rows)], pb.at[slot], sp[slot]),
        )

    def out_copy(k, slot):
        r0 = row_base + k * chunk_rows
        return pltpu.make_async_copy(xb.at[slot], o_hbm.at[pl.ds(r0, chunk_rows)], so[slot])

    for c in in_copies(0, 0):
        c.start()
    for k in range(n_chunks):
        slot = k % 2
        other = 1 - slot
        if k + 1 < n_chunks:
            if k >= 1:
                out_copy(k - 1, other).wait()
            for c in in_copies(k + 1, other):
                c.start()
        for c in in_copies(k, slot):
            c.wait()

        @plsc.parallel_loop(0, ce // LANES, unroll=8)
        def _add(i):
            r = i // (D_MODEL // LANES)
            c = (i % (D_MODEL // LANES)) * LANES
            sl = pl.ds(c, LANES)
            xb[slot, r, sl] = xb[slot, r, sl] + pb[slot, r, sl]

        out_copy(k, slot).start()
    if n_chunks >= 2:
        out_copy(n_chunks - 2, (n_chunks - 2) % 2).wait()
    out_copy(n_chunks - 1, (n_chunks - 1) % 2).wait()


def kernel(x, pe):
    B, S, D = x.shape
    rows = B * S
    pe_rows = pe.shape[0]
    chunk_rows = 32
    ce = chunk_rows * D

    mesh = plsc.VectorSubcoreMesh(core_axis_name="c", subcore_axis_name="s")
    sc = pl.kernel(
        functools.partial(_sc_add_kernel, rows=rows, pe_rows=pe_rows,
                          chunk_rows=chunk_rows),
        out_type=jax.ShapeDtypeStruct((rows, D), jnp.float32),
        mesh=mesh,
        scratch_types=[
            pltpu.VMEM((2, chunk_rows, D), jnp.float32),
            pltpu.VMEM((2, chunk_rows, D), jnp.float32),
        ] + [pltpu.SemaphoreType.DMA] * 6,
    )
    out = sc(x.reshape(rows, D), pe)
    return out.reshape(B, S, D)


# speedup vs baseline: 5.0286x; 1.3180x over previous
import functools

import jax
import jax.numpy as jnp
from jax import lax
from jax.experimental import pallas as pl
from jax.experimental.pallas import tpu as pltpu
from jax.experimental.pallas import tpu_sc as plsc

D_MODEL = 768
NUM_WORKERS = 32
LANES = 16


def _sc_add_kernel(x_hbm, pe_hbm, o_hbm, xb, pb,
                   sx0, sx1, sp0, sp1, so0, so1, *,
                   batch, pe_rows, chunk_rows):
    sx = (sx0, sx1)
    sp = (sp0, sp1)
    so = (so0, so1)
    w = lax.axis_index("s") * 2 + lax.axis_index("c")
    pe_band = pe_rows // NUM_WORKERS
    pe_base = w * pe_band
    n_chunks = pe_band // chunk_rows
    groups = D_MODEL // LANES

    def in_copies(k, slot):
        pr0 = pe_base + k * chunk_rows
        copies = [pltpu.make_async_copy(
            pe_hbm.at[pl.ds(pr0, chunk_rows)], pb.at[slot], sp[slot])]
        for b in range(batch):
            copies.append(pltpu.make_async_copy(
                x_hbm.at[pl.ds(b * pe_rows + pr0, chunk_rows)],
                xb.at[slot, b], sx[slot]))
        return copies

    def out_copies(k, slot):
        pr0 = pe_base + k * chunk_rows
        return [pltpu.make_async_copy(
            xb.at[slot, b], o_hbm.at[pl.ds(b * pe_rows + pr0, chunk_rows)],
            so[slot]) for b in range(batch)]

    for c in in_copies(0, 0):
        c.start()
    for k in range(n_chunks):
        slot = k % 2
        other = 1 - slot
        if k + 1 < n_chunks:
            if k >= 1:
                for c in out_copies(k - 1, other):
                    c.wait()
            for c in in_copies(k + 1, other):
                c.start()
        for c in in_copies(k, slot):
            c.wait()

        @plsc.parallel_loop(0, chunk_rows * groups, unroll=4)
        def _add(i):
            r = i // groups
            sl = pl.ds((i % groups) * LANES, LANES)
            pv = pb[slot, r, sl]
            for b in range(batch):
                xb[slot, b, r, sl] = xb[slot, b, r, sl] + pv

        for c in out_copies(k, slot):
            c.start()
    if n_chunks >= 2:
        for c in out_copies(n_chunks - 2, (n_chunks - 2) % 2):
            c.wait()
    for c in out_copies(n_chunks - 1, (n_chunks - 1) % 2):
        c.wait()


def kernel(x, pe):
    B, S, D = x.shape
    rows = B * S
    pe_rows = pe.shape[0]
    chunk_rows = 16

    mesh = plsc.VectorSubcoreMesh(core_axis_name="c", subcore_axis_name="s")
    sc = pl.kernel(
        functools.partial(_sc_add_kernel, batch=B, pe_rows=pe_rows,
                          chunk_rows=chunk_rows),
        out_type=jax.ShapeDtypeStruct((rows, D), jnp.float32),
        mesh=mesh,
        scratch_types=[
            pltpu.VMEM((2, B, chunk_rows, D), jnp.float32),
            pltpu.VMEM((2, chunk_rows, D), jnp.float32),
        ] + [pltpu.SemaphoreType.DMA] * 6,
    )
    out = sc(x.reshape(rows, D), pe)
    return out.reshape(B, S, D)


# speedup vs baseline: 5.5817x; 1.1100x over previous
import functools

import jax
import jax.numpy as jnp
from jax import lax
from jax.experimental import pallas as pl
from jax.experimental.pallas import tpu as pltpu
from jax.experimental.pallas import tpu_sc as plsc

D_MODEL = 768
NUM_WORKERS = 32
LANES = 16


def _sc_add_kernel(x_hbm, pe_hbm, o_hbm, xb, pb, ob,
                   sx0, sx1, sp0, sp1, so0, so1, *,
                   batch, pe_rows, chunk_rows):
    sx = (sx0, sx1)
    sp = (sp0, sp1)
    so = (so0, so1)
    w = lax.axis_index("s") * 2 + lax.axis_index("c")
    pe_band = pe_rows // NUM_WORKERS
    pe_base = w * pe_band
    n_chunks = pe_band // chunk_rows
    groups = D_MODEL // LANES

    def in_copies(k, slot):
        pr0 = pe_base + k * chunk_rows
        copies = [pltpu.make_async_copy(
            pe_hbm.at[pl.ds(pr0, chunk_rows)], pb.at[slot], sp[slot])]
        for b in range(batch):
            copies.append(pltpu.make_async_copy(
                x_hbm.at[pl.ds(b * pe_rows + pr0, chunk_rows)],
                xb.at[slot, b], sx[slot]))
        return copies

    def out_copies(k, slot):
        pr0 = pe_base + k * chunk_rows
        return [pltpu.make_async_copy(
            ob.at[slot, b], o_hbm.at[pl.ds(b * pe_rows + pr0, chunk_rows)],
            so[slot]) for b in range(batch)]

    for c in in_copies(0, 0):
        c.start()
    for k in range(n_chunks):
        slot = k % 2
        other = 1 - slot
        if k + 1 < n_chunks:
            for c in in_copies(k + 1, other):
                c.start()
        if k >= 2:
            for c in out_copies(k - 2, slot):
                c.wait()
        for c in in_copies(k, slot):
            c.wait()

        @plsc.parallel_loop(0, chunk_rows * groups, unroll=4)
        def _add(i):
            r = i // groups
            sl = pl.ds((i % groups) * LANES, LANES)
            pv = pb[slot, r, sl]
            for b in range(batch):
                ob[slot, b, r, sl] = xb[slot, b, r, sl] + pv

        for c in out_copies(k, slot):
            c.start()
    for k in (n_chunks - 2, n_chunks - 1):
        if k >= 0:
            for c in out_copies(k, k % 2):
                c.wait()


def kernel(x, pe):
    B, S, D = x.shape
    rows = B * S
    pe_rows = pe.shape[0]
    chunk_rows = 8

    mesh = plsc.VectorSubcoreMesh(core_axis_name="c", subcore_axis_name="s")
    sc = pl.kernel(
        functools.partial(_sc_add_kernel, batch=B, pe_rows=pe_rows,
                          chunk_rows=chunk_rows),
        out_type=jax.ShapeDtypeStruct((rows, D), jnp.float32),
        mesh=mesh,
        scratch_types=[
            pltpu.VMEM((2, B, chunk_rows, D), jnp.float32),
            pltpu.VMEM((2, chunk_rows, D), jnp.float32),
            pltpu.VMEM((2, B, chunk_rows, D), jnp.float32),
        ] + [pltpu.SemaphoreType.DMA] * 6,
    )
    out = sc(x.reshape(rows, D), pe)
    return out.reshape(B, S, D)
